# baseline (speedup 1.0000x reference)
import functools

import jax
import jax.numpy as jnp
from jax import lax
from jax.experimental import pallas as pl
from jax.experimental.pallas import tpu as pltpu
from jax.experimental.pallas import tpu_sc as plsc

NENTITY = 1000000
NRELATION = 1000
D = 64
B = 16384
L = 16
NC, NS = 2, 16
NW = NC * NS
BPW = B // NW
CHUNK = 128
NCHUNK = BPW // CHUNK
GROUPS = BPW // L


def _sc_score(heads, rels, tails, ent, rel, gamma_arr):
    mesh = plsc.VectorSubcoreMesh(
        core_axis_name="c", subcore_axis_name="s", num_cores=NC, num_subcores=NS
    )

    @functools.partial(
        pl.kernel,
        out_type=jax.ShapeDtypeStruct((B,), jnp.float32),
        mesh=mesh,
        compiler_params=pltpu.CompilerParams(
            needs_layout_passes=False, use_tc_tiling_on_sc=False
        ),
        scratch_types=dict(
            idx_h=pltpu.VMEM((NCHUNK, CHUNK), jnp.int32),
            idx_r=pltpu.VMEM((NCHUNK, CHUNK), jnp.int32),
            idx_t=pltpu.VMEM((NCHUNK, CHUNK), jnp.int32),
            h_v=pltpu.VMEM((BPW, D), jnp.float32),
            r_v=pltpu.VMEM((BPW, D), jnp.float32),
            t_v=pltpu.VMEM((BPW, D), jnp.float32),
            out_v=pltpu.VMEM((BPW,), jnp.float32),
            tsp=pltpu.VMEM((L * L,), jnp.float32),
            gamma_v=pltpu.VMEM((L,), jnp.float32),
            sem=pltpu.SemaphoreType.DMA,
        ),
    )
    def body(heads_hbm, rels_hbm, tails_hbm, ent_hbm, rel_hbm, gamma_hbm,
             out_hbm, idx_h, idx_r, idx_t, h_v, r_v, t_v, out_v, tsp,
             gamma_v, sem):
        wid = lax.axis_index("s") * NC + lax.axis_index("c")
        iblk = wid * NCHUNK

        pltpu.sync_copy(heads_hbm.at[pl.ds(iblk, NCHUNK)], idx_h)
        pltpu.sync_copy(rels_hbm.at[pl.ds(iblk, NCHUNK)], idx_r)
        pltpu.sync_copy(tails_hbm.at[pl.ds(iblk, NCHUNK)], idx_t)
        pltpu.sync_copy(gamma_hbm, gamma_v)

        cps = []
        for j in range(NCHUNK):
            dst = pl.ds(j * CHUNK, CHUNK)
            cps.append(pltpu.async_copy(ent_hbm.at[idx_h.at[j]], h_v.at[dst], sem))
            cps.append(pltpu.async_copy(rel_hbm.at[idx_r.at[j]], r_v.at[dst], sem))
            cps.append(pltpu.async_copy(ent_hbm.at[idx_t.at[j]], t_v.at[dst], sem))
        for cp in cps:
            cp.wait()

        lane = lax.iota(jnp.int32, L)
        gvec = gamma_v[...]

        def group_body(g, carry):
            for r in range(L):
                row = g * L + r
                acc = None
                for c in range(D // L):
                    dsl = pl.ds(c * L, L)
                    term = jnp.abs(h_v[row, dsl] + r_v[row, dsl] - t_v[row, dsl])
                    acc = term if acc is None else acc + term
                plsc.store_scatter(tsp, [lane * L + r], acc)
            tot = tsp[pl.ds(0, L)]
            for j in range(1, L):
                tot = tot + tsp[pl.ds(j * L, L)]
            out_v[pl.ds(g * L, L)] = gvec - tot
            return carry

        lax.fori_loop(0, GROUPS, group_body, 0)

        pltpu.sync_copy(out_v, out_hbm.at[pl.ds(wid * BPW, BPW)])

    return body(heads, rels, tails, ent, rel, gamma_arr)


def kernel(sample, entity_embedding, relation_embedding, gamma):
    heads = sample[:, 0].reshape(B // CHUNK, CHUNK)
    rels = sample[:, 1].reshape(B // CHUNK, CHUNK)
    tails = sample[:, 2].reshape(B // CHUNK, CHUNK)
    gamma_arr = jnp.full((L,), gamma, dtype=jnp.float32)
    score = _sc_score(heads, rels, tails, entity_embedding,
                      relation_embedding, gamma_arr)
    return score.reshape(B, 1)


# speedup vs baseline: 1.5142x; 1.5142x over previous
import functools

import jax
import jax.numpy as jnp
from jax import lax
from jax.experimental import pallas as pl
from jax.experimental.pallas import tpu as pltpu
from jax.experimental.pallas import tpu_sc as plsc

NENTITY = 1000000
NRELATION = 1000
D = 64
B = 16384
L = 16
NC, NS = 2, 16
NW = NC * NS
BPW = B // NW
C = 16
NCHUNK = BPW // C
NBUF = 2


def _sc_score(heads, rels, tails, ent, relp, gamma_arr):
    mesh = plsc.VectorSubcoreMesh(
        core_axis_name="c", subcore_axis_name="s", num_cores=NC, num_subcores=NS
    )

    @functools.partial(
        pl.kernel,
        out_type=jax.ShapeDtypeStruct((B,), jnp.float32),
        mesh=mesh,
        compiler_params=pltpu.CompilerParams(needs_layout_passes=False),
        scratch_types=dict(
            r_ids=pltpu.VMEM((BPW,), jnp.int32),
            h_idv=pltpu.VMEM((BPW,), jnp.int32),
            t_idv=pltpu.VMEM((BPW,), jnp.int32),
            h_rows=pltpu.VMEM((NBUF, C, D), jnp.float32),
            t_rows=pltpu.VMEM((NBUF, C, D), jnp.float32),
            r_rows=pltpu.VMEM((NBUF, C, 2 * D), jnp.float32),
            out_v=pltpu.VMEM((BPW,), jnp.float32),
            gamma_v=pltpu.VMEM((L,), jnp.float32),
            sem0=pltpu.SemaphoreType.DMA,
            sem1=pltpu.SemaphoreType.DMA,
        ),
    )
    def body(heads_hbm, rels_hbm, tails_hbm, ent_hbm, rel_hbm, gamma_hbm,
             out_hbm, r_ids, h_idv, t_idv, h_rows, t_rows,
             r_rows, out_v, gamma_v, sem0, sem1):
        wid = lax.axis_index("s") * NC + lax.axis_index("c")
        base = wid * BPW
        sems = (sem0, sem1)

        pltpu.sync_copy(heads_hbm.at[pl.ds(base, BPW)], h_idv)
        pltpu.sync_copy(tails_hbm.at[pl.ds(base, BPW)], t_idv)
        pltpu.sync_copy(rels_hbm.at[pl.ds(base, BPW)], r_ids)
        pltpu.sync_copy(gamma_hbm, gamma_v)

        def issue(c, buf):
            sem = sems[buf]
            hv16 = h_idv[pl.ds(c * C, C)]
            tv16 = t_idv[pl.ds(c * C, C)]
            for j in range(C):
                h = hv16[j]
                t = tv16[j]
                pltpu.async_copy(ent_hbm.at[pl.ds(h, 1)],
                                 h_rows.at[buf].at[pl.ds(j, 1)], sem)
                pltpu.async_copy(ent_hbm.at[pl.ds(t, 1)],
                                 t_rows.at[buf].at[pl.ds(j, 1)], sem)
            pltpu.async_copy(rel_hbm.at[r_ids.at[pl.ds(c * C, C)]],
                             r_rows.at[buf], sem)

        def drain(c, buf):
            sem = sems[buf]
            for j in range(C):
                pltpu.make_async_copy(ent_hbm.at[pl.ds(0, 1)],
                                      h_rows.at[buf].at[pl.ds(j, 1)], sem).wait()
                pltpu.make_async_copy(ent_hbm.at[pl.ds(0, 1)],
                                      t_rows.at[buf].at[pl.ds(j, 1)], sem).wait()
            pltpu.make_async_copy(rel_hbm.at[r_ids.at[pl.ds(c * C, C)]],
                                  r_rows.at[buf], sem).wait()

        issue(0, 0)
        issue(1, 1)

        lane = lax.iota(jnp.int32, L)
        gvec = gamma_v[...]

        def chunk_body(half, carry):
            for b in range(NBUF):
                c = half * NBUF + b
                drain(c, b)
                acc = None
                for e in range(D):
                    ev = jnp.full((L,), e, jnp.int32)
                    hv = plsc.load_gather(h_rows.at[b], [lane, ev])
                    tv = plsc.load_gather(t_rows.at[b], [lane, ev])
                    rv = plsc.load_gather(r_rows.at[b], [lane, ev])
                    term = jnp.abs(hv + rv - tv)
                    acc = term if acc is None else acc + term
                out_v[pl.ds(c * C, C)] = gvec - acc

                @pl.when(c + NBUF < NCHUNK)
                def _():
                    issue(c + NBUF, b)

            return carry

        lax.fori_loop(0, NCHUNK // NBUF, chunk_body, 0)

        pltpu.sync_copy(out_v, out_hbm.at[pl.ds(base, BPW)])

    return body(heads, rels, tails, ent, relp, gamma_arr)


def kernel(sample, entity_embedding, relation_embedding, gamma):
    heads = sample[:, 0]
    rels = sample[:, 1]
    tails = sample[:, 2]
    relp = jnp.pad(relation_embedding, ((0, 0), (0, D)))
    gamma_arr = jnp.full((L,), gamma, dtype=jnp.float32)
    score = _sc_score(heads, rels, tails, entity_embedding, relp, gamma_arr)
    return score.reshape(B, 1)
